# baseline (speedup 1.0000x reference)
import functools

import jax
import jax.numpy as jnp
from jax import lax
from jax.experimental import pallas as pl
from jax.experimental.pallas import tpu as pltpu
from jax.experimental.pallas import tpu_sc as plsc

NC, NS, LANES = 2, 16, 16
NW = NC * NS
B = 4096
DIM = 64
NEG = 20
BPW = B // NW
CHUNK = 128
NCHUNK = BPW * NEG // CHUNK


def _sc_gather(Wv, Wu, cidx, tidx, nidx, grp):
    mesh = plsc.VectorSubcoreMesh(core_axis_name="c", subcore_axis_name="s")
    out_t = [jax.ShapeDtypeStruct((B, DIM), jnp.float32)] * 3

    @functools.partial(
        pl.kernel,
        out_type=out_t,
        mesh=mesh,
        compiler_params=pltpu.CompilerParams(use_tc_tiling_on_sc=False),
        scratch_types=[
            pltpu.VMEM((BPW,), jnp.int32),
            pltpu.VMEM((BPW,), jnp.int32),
            pltpu.VMEM((NCHUNK, CHUNK), jnp.int32),
            pltpu.VMEM((NCHUNK, CHUNK), jnp.int32),
            pltpu.VMEM((BPW, DIM), jnp.float32),
            pltpu.VMEM((CHUNK, DIM), jnp.float32),
            pltpu.VMEM_SHARED((NS * BPW, DIM), jnp.float32),
        ],
    )
    def k(wv_hbm, wu_hbm, c_hbm, t_hbm, n_hbm, g_hbm,
          oc_hbm, ot_hbm, on_hbm,
          civ, tiv, niv, giv, rows, grows, acc):
        sid = lax.axis_index("s")
        wid = lax.axis_index("c") * NS + sid
        base = wid * BPW

        pltpu.sync_copy(c_hbm.at[pl.ds(base, BPW)], civ)
        pltpu.sync_copy(wv_hbm.at[civ], rows)
        pltpu.sync_copy(rows, oc_hbm.at[pl.ds(base, BPW)])

        pltpu.sync_copy(t_hbm.at[pl.ds(base, BPW)], tiv)
        pltpu.sync_copy(wu_hbm.at[tiv], rows)
        pltpu.sync_copy(rows, ot_hbm.at[pl.ds(base, BPW)])

        zero = jnp.zeros((LANES,), jnp.float32)

        @pl.loop(0, BPW)
        def _(i):
            @pl.loop(0, DIM, step=LANES)
            def _(c0):
                rows[i, pl.ds(c0, LANES)] = zero

        pltpu.sync_copy(rows, acc.at[pl.ds(sid * BPW, BPW)])

        pltpu.sync_copy(n_hbm.at[wid], niv)
        pltpu.sync_copy(g_hbm.at[sid], giv)

        @pl.loop(0, NCHUNK)
        def _(j):
            pltpu.sync_copy(wu_hbm.at[niv.at[j]], grows)
            pltpu.sync_copy(grows, acc.at[giv.at[j]], add=True)

        pltpu.sync_copy(acc.at[pl.ds(sid * BPW, BPW)], on_hbm.at[pl.ds(base, BPW)])

    return k(Wv, Wu, cidx, tidx, nidx, grp)


def _tc_loss(ce, te, ns):
    def body(c_ref, t_ref, n_ref, o_ref):
        c = c_ref[...]
        t = t_ref[...]
        n = n_ref[...]
        pos = jnp.sum(c * t, axis=1)
        neg = -jnp.sum(c * n, axis=1)

        def logsig(x):
            return jnp.minimum(x, 0.0) - jnp.log1p(jnp.exp(-jnp.abs(x)))

        tot = jnp.sum(logsig(pos)) + jnp.sum(logsig(neg))
        o_ref[...] = jnp.reshape(-tot / B, (1, 1))

    return pl.pallas_call(
        body,
        out_shape=jax.ShapeDtypeStruct((1, 1), jnp.float32),
    )(ce, te, ns)


def kernel(center_words, target_words, negative_words, Wv, Wu):
    nidx = negative_words.reshape(NW, NCHUNK, CHUNK)
    local_grp = jnp.arange(BPW * NEG, dtype=jnp.int32) // NEG
    grp = (jnp.arange(NS, dtype=jnp.int32)[:, None] * BPW
           + local_grp[None, :]).reshape(NS, NCHUNK, CHUNK)
    ce, te, nsum = _sc_gather(Wv, Wu, center_words, target_words, nidx, grp)
    out = _tc_loss(ce, te, nsum)
    return jnp.reshape(out, ())


# speedup vs baseline: 1.0137x; 1.0137x over previous
import functools

import jax
import jax.numpy as jnp
from jax import lax
from jax.experimental import pallas as pl
from jax.experimental.pallas import tpu as pltpu
from jax.experimental.pallas import tpu_sc as plsc

NC, NS, LANES = 2, 16, 16
NW = NC * NS
B = 4096
DIM = 64
NEG = 20
BPW = B // NW
NBUF = 4


def _sc_gather(Wv, Wu, cidx, tidx, nidx):
    mesh = plsc.VectorSubcoreMesh(core_axis_name="c", subcore_axis_name="s")
    out_t = [jax.ShapeDtypeStruct((B, DIM), jnp.float32)] * 3

    @functools.partial(
        pl.kernel,
        out_type=out_t,
        mesh=mesh,
        compiler_params=pltpu.CompilerParams(use_tc_tiling_on_sc=False),
        scratch_types=[
            pltpu.VMEM((BPW,), jnp.int32),
            pltpu.VMEM((BPW,), jnp.int32),
            pltpu.VMEM((NEG, BPW), jnp.int32),
            pltpu.VMEM((BPW, DIM), jnp.float32),
            pltpu.VMEM((BPW, DIM), jnp.float32),
            pltpu.VMEM((BPW, DIM), jnp.float32),
        ]
        + [pltpu.VMEM((BPW, DIM), jnp.float32)] * NBUF
        + [pltpu.SemaphoreType.DMA] * (NBUF + 2),
    )
    def k(wv_hbm, wu_hbm, c_hbm, t_hbm, n_hbm,
          oc_hbm, ot_hbm, on_hbm,
          civ, tiv, niv, cbuf, tbuf, acc, nb0, nb1, nb2, nb3,
          s0, s1, s2, s3, sc, st):
        sid = lax.axis_index("s")
        wid = lax.axis_index("c") * NS + sid
        base = wid * BPW

        pltpu.sync_copy(c_hbm.at[pl.ds(base, BPW)], civ)
        pltpu.sync_copy(t_hbm.at[pl.ds(base, BPW)], tiv)
        pltpu.sync_copy(n_hbm.at[wid], niv)

        fc = pltpu.async_copy(wv_hbm.at[civ], cbuf, sc)
        ft = pltpu.async_copy(wu_hbm.at[tiv], tbuf, st)

        nbufs = [nb0, nb1, nb2, nb3]
        sems = [s0, s1, s2, s3]
        pend = [
            pltpu.async_copy(wu_hbm.at[niv.at[j]], nbufs[j], sems[j])
            for j in range(NBUF)
        ]
        for j in range(NEG):
            b = j % NBUF
            pend[b].wait()
            buf = nbufs[b]
            if j == 0:
                @pl.loop(0, BPW)
                def _(i, buf=buf):
                    for c0 in range(0, DIM, LANES):
                        acc[i, pl.ds(c0, LANES)] = buf[i, pl.ds(c0, LANES)]
            else:
                @pl.loop(0, BPW)
                def _(i, buf=buf):
                    for c0 in range(0, DIM, LANES):
                        plsc.addupdate(acc.at[i, pl.ds(c0, LANES)],
                                       buf[i, pl.ds(c0, LANES)])
            nxt = j + NBUF
            if nxt < NEG:
                pend[b] = pltpu.async_copy(wu_hbm.at[niv.at[nxt]], nbufs[b],
                                           sems[b])

        fc.wait()
        ft.wait()
        pltpu.sync_copy(cbuf, oc_hbm.at[pl.ds(base, BPW)])
        pltpu.sync_copy(tbuf, ot_hbm.at[pl.ds(base, BPW)])
        pltpu.sync_copy(acc, on_hbm.at[pl.ds(base, BPW)])

    return k(Wv, Wu, cidx, tidx, nidx)


def _tc_loss(ce, te, ns):
    def body(c_ref, t_ref, n_ref, o_ref):
        c = c_ref[...]
        t = t_ref[...]
        n = n_ref[...]
        pos = jnp.sum(c * t, axis=1)
        neg = -jnp.sum(c * n, axis=1)

        def logsig(x):
            return jnp.minimum(x, 0.0) - jnp.log1p(jnp.exp(-jnp.abs(x)))

        tot = jnp.sum(logsig(pos)) + jnp.sum(logsig(neg))
        o_ref[...] = jnp.reshape(-tot / B, (1, 1))

    return pl.pallas_call(
        body,
        out_shape=jax.ShapeDtypeStruct((1, 1), jnp.float32),
    )(ce, te, ns)


def kernel(center_words, target_words, negative_words, Wv, Wu):
    nidx = jnp.transpose(negative_words.reshape(NW, BPW, NEG), (0, 2, 1))
    ce, te, nsum = _sc_gather(Wv, Wu, center_words, target_words, nidx)
    out = _tc_loss(ce, te, nsum)
    return jnp.reshape(out, ())
